# baseline (speedup 1.0000x reference)
import jax
import jax.numpy as jnp
from jax.experimental import pallas as pl

_TN = 512
_K = 1024
_D = 256


def _dot(a, b, precision=jax.lax.Precision.DEFAULT):
    return jax.lax.dot_general(
        a, b, dimension_numbers=(((1,), (0,)), ((), ())),
        preferred_element_type=jnp.float32, precision=precision)


def _rq_kernel(x_ref, cb0t_ref, cb1t_ref, cb2t_ref,
               h0_ref, m0_ref, l0_ref, h1_ref, m1_ref, l1_ref,
               o0_ref, o1_ref, o2_ref):
    r = x_ref[...]
    iota = jax.lax.broadcasted_iota(jnp.int32, (r.shape[0], _K), 1)

    def layer(r, cbt_ref, split_refs, o_ref):
        cbt = cbt_ref[...]
        cn = jnp.sum(cbt * cbt, axis=0, keepdims=True)
        g = _dot(r, cbt)
        s = cn - 2.0 * g
        m = jnp.min(s, axis=1, keepdims=True)
        idx = jnp.min(jnp.where(s == m, iota, _K), axis=1, keepdims=True)
        o_ref[...] = idx
        if split_refs is None:
            return r
        onehot = (iota == idx).astype(jnp.bfloat16)
        hr, mr, lr = split_refs
        q = _dot(onehot, hr[...]) + _dot(onehot, mr[...]) + _dot(onehot, lr[...])
        return r - q

    r = layer(r, cb0t_ref, (h0_ref, m0_ref, l0_ref), o0_ref)
    r = layer(r, cb1t_ref, (h1_ref, m1_ref, l1_ref), o1_ref)
    layer(r, cb2t_ref, None, o2_ref)


def _split3(cb):
    hi = jax.lax.optimization_barrier(cb.astype(jnp.bfloat16))
    rem = cb - hi.astype(jnp.float32)
    mid = jax.lax.optimization_barrier(rem.astype(jnp.bfloat16))
    lo = (rem - mid.astype(jnp.float32)).astype(jnp.bfloat16)
    return hi, mid, lo


def kernel(x, cb0, cb1, cb2):
    n, d = x.shape
    full = pl.BlockSpec((d, _K), lambda i: (0, 0))
    split = pl.BlockSpec((_K, d), lambda i: (0, 0))
    outs = pl.pallas_call(
        _rq_kernel,
        grid=(n // _TN,),
        in_specs=[
            pl.BlockSpec((_TN, d), lambda i: (i, 0)),
            full, full, full,
            split, split, split, split, split, split,
        ],
        out_specs=[pl.BlockSpec((_TN, 1), lambda i: (i, 0))] * 3,
        out_shape=[jax.ShapeDtypeStruct((n, 1), jnp.int32)] * 3,
    )(x, cb0.T, cb1.T, cb2.T, *_split3(cb0), *_split3(cb1))
    return jnp.concatenate(outs, axis=1)


# speedup vs baseline: 1.0840x; 1.0840x over previous
import jax
import jax.numpy as jnp
from jax.experimental import pallas as pl
from jax.experimental.pallas import tpu as pltpu

_TN = 1024
_K = 1024
_D = 256


def _dot(a, b):
    return jax.lax.dot_general(
        a, b, dimension_numbers=(((1,), (0,)), ((), ())),
        preferred_element_type=jnp.float32,
        precision=jax.lax.Precision.DEFAULT)


def _rq_kernel(x_ref, cb0t_ref, cb1t_ref, cb2t_ref,
               h0_ref, m0_ref, l0_ref, h1_ref, m1_ref, l1_ref,
               o0_ref, o1_ref, o2_ref, cn_ref):
    @pl.when(pl.program_id(0) == 0)
    def _():
        for row, ref in ((0, cb0t_ref), (1, cb1t_ref), (2, cb2t_ref)):
            cbt = ref[...]
            cn_ref[row, :] = jnp.sum(cbt * cbt, axis=0) * 0.25

    r = x_ref[...]
    iota = jax.lax.broadcasted_iota(jnp.int32, (r.shape[0], _K), 1)

    def layer(r, row, cbt_ref, split_refs, o_ref):
        g2 = _dot(r, cbt_ref[...])
        s = cn_ref[row:row + 1, :] + g2
        m = jnp.min(s, axis=1, keepdims=True)
        idx = jnp.min(jnp.where(s == m, iota, _K), axis=1, keepdims=True)
        o_ref[...] = idx
        if split_refs is None:
            return r
        onehot = (iota == idx).astype(jnp.bfloat16)
        hr, mr, lr = split_refs
        q = _dot(onehot, hr[...]) + _dot(onehot, mr[...]) + _dot(onehot, lr[...])
        return r - q

    r = layer(r, 0, cb0t_ref, (h0_ref, m0_ref, l0_ref), o0_ref)
    r = layer(r, 1, cb1t_ref, (h1_ref, m1_ref, l1_ref), o1_ref)
    layer(r, 2, cb2t_ref, None, o2_ref)


def _split3(cb):
    hi = jax.lax.optimization_barrier(cb.astype(jnp.bfloat16))
    rem = cb - hi.astype(jnp.float32)
    mid = jax.lax.optimization_barrier(rem.astype(jnp.bfloat16))
    lo = (rem - mid.astype(jnp.float32)).astype(jnp.bfloat16)
    return hi, mid, lo


def kernel(x, cb0, cb1, cb2):
    n, d = x.shape
    full = pl.BlockSpec((d, _K), lambda i: (0, 0))
    split = pl.BlockSpec((_K, d), lambda i: (0, 0))
    outs = pl.pallas_call(
        _rq_kernel,
        grid=(n // _TN,),
        in_specs=[
            pl.BlockSpec((_TN, d), lambda i: (i, 0)),
            full, full, full,
            split, split, split, split, split, split,
        ],
        out_specs=[pl.BlockSpec((_TN, 1), lambda i: (i, 0))] * 3,
        out_shape=[jax.ShapeDtypeStruct((n, 1), jnp.int32)] * 3,
        scratch_shapes=[pltpu.VMEM((8, _K), jnp.float32)],
    )(x, (-2.0 * cb0).T, (-2.0 * cb1).T, (-2.0 * cb2).T,
      *_split3(cb0), *_split3(cb1))
    return jnp.concatenate(outs, axis=1)


# speedup vs baseline: 1.0873x; 1.0031x over previous
import jax
import jax.numpy as jnp
from jax.experimental import pallas as pl
from jax.experimental.pallas import tpu as pltpu

_TN = 1024
_K = 1024
_D = 256


def _dot(a, b):
    return jax.lax.dot_general(
        a, b, dimension_numbers=(((1,), (0,)), ((), ())),
        preferred_element_type=jnp.float32,
        precision=jax.lax.Precision.DEFAULT)


def _rq_kernel(x_ref, cb0t_ref, cb1t_ref, cb2t_ref,
               h0_ref, m0_ref, l0_ref, h1_ref, m1_ref, l1_ref,
               o0_ref, o1_ref, o2_ref, cn_ref):
    @pl.when(pl.program_id(0) == 0)
    def _():
        for row, ref in ((0, cb0t_ref), (1, cb1t_ref), (2, cb2t_ref)):
            cbt = ref[...]
            cn_ref[row, :] = jnp.sum(cbt * cbt, axis=0) * 0.25

    half = _TN // 2
    iota = jax.lax.broadcasted_iota(jnp.int32, (half, _K), 1).astype(jnp.float32)

    def layer(r, row, cbt_ref, split_refs, o_ref, lo_half):
        g2 = _dot(r, cbt_ref[...])
        s = cn_ref[row:row + 1, :] + g2
        m = jnp.min(s, axis=1, keepdims=True)
        idx = jnp.min(jnp.where(s == m, iota, float(_K)), axis=1, keepdims=True)
        sl = slice(0, half) if lo_half else slice(half, _TN)
        o_ref[sl, :] = idx.astype(jnp.int32)
        if split_refs is None:
            return r
        onehot = (iota == idx).astype(jnp.bfloat16)
        hr, mr, lr = split_refs
        q = _dot(onehot, hr[...]) + _dot(onehot, mr[...]) + _dot(onehot, lr[...])
        return r - q

    for lo_half in (True, False):
        r = x_ref[slice(0, half) if lo_half else slice(half, _TN), :]
        r = layer(r, 0, cb0t_ref, (h0_ref, m0_ref, l0_ref), o0_ref, lo_half)
        r = layer(r, 1, cb1t_ref, (h1_ref, m1_ref, l1_ref), o1_ref, lo_half)
        layer(r, 2, cb2t_ref, None, o2_ref, lo_half)


def _split3(cb):
    hi = jax.lax.optimization_barrier(cb.astype(jnp.bfloat16))
    rem = cb - hi.astype(jnp.float32)
    mid = jax.lax.optimization_barrier(rem.astype(jnp.bfloat16))
    lo = (rem - mid.astype(jnp.float32)).astype(jnp.bfloat16)
    return hi, mid, lo


def kernel(x, cb0, cb1, cb2):
    n, d = x.shape
    full = pl.BlockSpec((d, _K), lambda i: (0, 0))
    split = pl.BlockSpec((_K, d), lambda i: (0, 0))
    outs = pl.pallas_call(
        _rq_kernel,
        grid=(n // _TN,),
        in_specs=[
            pl.BlockSpec((_TN, d), lambda i: (i, 0)),
            full, full, full,
            split, split, split, split, split, split,
        ],
        out_specs=[pl.BlockSpec((_TN, 1), lambda i: (i, 0))] * 3,
        out_shape=[jax.ShapeDtypeStruct((n, 1), jnp.int32)] * 3,
        scratch_shapes=[pltpu.VMEM((8, _K), jnp.float32)],
    )(x, (-2.0 * cb0).T, (-2.0 * cb1).T, (-2.0 * cb2).T,
      *_split3(cb0), *_split3(cb1))
    return jnp.concatenate(outs, axis=1)


# speedup vs baseline: 1.1301x; 1.0393x over previous
import jax
import jax.numpy as jnp
from jax.experimental import pallas as pl
from jax.experimental.pallas import tpu as pltpu

_TN = 1024
_K = 1024
_D = 256


def _dot(a, b):
    return jax.lax.dot_general(
        a, b, dimension_numbers=(((1,), (0,)), ((), ())),
        preferred_element_type=jnp.float32,
        precision=jax.lax.Precision.DEFAULT)


def _rq_kernel(x_ref, cb0t_ref, cb1t_ref, cb2t_ref,
               h0_ref, m0_ref, l0_ref, h1_ref, m1_ref, l1_ref,
               o0_ref, o1_ref, o2_ref, cn_ref):
    @pl.when(pl.program_id(0) == 0)
    def _():
        for row, ref in ((0, cb0t_ref), (1, cb1t_ref), (2, cb2t_ref)):
            cbt = ref[...]
            cn_ref[row, :] = jnp.sum(cbt * cbt, axis=0) * 0.25

    half = _TN // 2
    iota = jax.lax.broadcasted_iota(jnp.int32, (half, _K), 1).astype(jnp.float32)

    def layer(r, row, cbt_ref, split_refs, o_ref, lo_half):
        g2 = _dot(r, cbt_ref[...])
        s = cn_ref[row:row + 1, :] + g2
        m = jnp.min(s, axis=1, keepdims=True)
        mask = s == m
        idx = jnp.min(jnp.where(mask, iota, float(_K)), axis=1, keepdims=True)
        sl = slice(0, half) if lo_half else slice(half, _TN)
        o_ref[sl, :] = idx.astype(jnp.int32)
        if split_refs is None:
            return r
        onehot = mask.astype(jnp.bfloat16)
        hr, mr, lr = split_refs
        q = _dot(onehot, hr[...]) + _dot(onehot, mr[...]) + _dot(onehot, lr[...])
        return r - q

    for lo_half in (True, False):
        r = x_ref[slice(0, half) if lo_half else slice(half, _TN), :]
        r = layer(r, 0, cb0t_ref, (h0_ref, m0_ref, l0_ref), o0_ref, lo_half)
        r = layer(r, 1, cb1t_ref, (h1_ref, m1_ref, l1_ref), o1_ref, lo_half)
        layer(r, 2, cb2t_ref, None, o2_ref, lo_half)


def _split3(cb):
    hi = jax.lax.optimization_barrier(cb.astype(jnp.bfloat16))
    rem = cb - hi.astype(jnp.float32)
    mid = jax.lax.optimization_barrier(rem.astype(jnp.bfloat16))
    lo = (rem - mid.astype(jnp.float32)).astype(jnp.bfloat16)
    return hi, mid, lo


def kernel(x, cb0, cb1, cb2):
    n, d = x.shape
    full = pl.BlockSpec((d, _K), lambda i: (0, 0))
    split = pl.BlockSpec((_K, d), lambda i: (0, 0))
    outs = pl.pallas_call(
        _rq_kernel,
        grid=(n // _TN,),
        in_specs=[
            pl.BlockSpec((_TN, d), lambda i: (i, 0)),
            full, full, full,
            split, split, split, split, split, split,
        ],
        out_specs=[pl.BlockSpec((_TN, 1), lambda i: (i, 0))] * 3,
        out_shape=[jax.ShapeDtypeStruct((n, 1), jnp.int32)] * 3,
        scratch_shapes=[pltpu.VMEM((8, _K), jnp.float32)],
    )(x, (-2.0 * cb0).T, (-2.0 * cb1).T, (-2.0 * cb2).T,
      *_split3(cb0), *_split3(cb1))
    return jnp.concatenate(outs, axis=1)
